# baseline (speedup 1.0000x reference)
import jax
import jax.numpy as jnp
from jax.experimental import pallas as pl
from jax.experimental.pallas import tpu as pltpu

N_TOKENS = 8192
HIDDEN = 2048
N_EXPERTS = 16
TOPK = 2
AUX_COEF = 0.001
Z_COEF = 0.001
BLK = 512


def _router_kernel(x_ref, wt_ref, w_out, i_out, aux_out,
                   cnt_ref, psum_ref, zsum_ref):
    step = pl.program_id(0)
    nsteps = pl.num_programs(0)

    @pl.when(step == 0)
    def _init():
        cnt_ref[...] = jnp.zeros_like(cnt_ref)
        psum_ref[...] = jnp.zeros_like(psum_ref)
        zsum_ref[0, 0] = 0.0

    logits = jnp.dot(x_ref[...], wt_ref[...],
                     preferred_element_type=jnp.float32)
    iota = jax.lax.broadcasted_iota(jnp.int32, logits.shape, 1)

    m1 = jnp.max(logits, axis=1, keepdims=True)
    i1 = jnp.min(jnp.where(logits == m1, iota, N_EXPERTS),
                 axis=1, keepdims=True)
    masked = jnp.where(iota == i1, -jnp.inf, logits)
    m2 = jnp.max(masked, axis=1, keepdims=True)
    i2 = jnp.min(jnp.where(masked == m2, iota, N_EXPERTS),
                 axis=1, keepdims=True)

    ex = jnp.exp(logits - m1)
    denom = jnp.sum(ex, axis=1, keepdims=True)
    probs = ex / denom
    sel1 = iota == i1
    sel2 = iota == i2
    p1 = jnp.sum(jnp.where(sel1, probs, 0.0), axis=1, keepdims=True)
    p2 = jnp.sum(jnp.where(sel2, probs, 0.0), axis=1, keepdims=True)
    tot = p1 + p2
    w_out[...] = jnp.concatenate([p1 / tot, p2 / tot], axis=1)
    i_out[...] = jnp.concatenate([i1, i2], axis=1)

    cnt_ref[...] += jnp.sum(sel1.astype(jnp.float32)
                            + sel2.astype(jnp.float32),
                            axis=0, keepdims=True)
    psum_ref[...] += jnp.sum(probs, axis=0, keepdims=True)
    log_z = m1 + jnp.log(denom)
    zsum_ref[0, 0] += jnp.sum(log_z * log_z)

    @pl.when(step == nsteps - 1)
    def _fin():
        f = cnt_ref[...] / (N_TOKENS * TOPK)
        p_mean = psum_ref[...] / N_TOKENS
        lb_loss = N_EXPERTS * jnp.sum(f * p_mean)
        z_loss = zsum_ref[0, 0] / N_TOKENS
        aux_out[0, 0] = AUX_COEF * lb_loss + Z_COEF * z_loss


@jax.jit
def kernel(hidden_states, gate_weight):
    wt = gate_weight.T
    grid = (N_TOKENS // BLK,)
    weights, indices, aux = pl.pallas_call(
        _router_kernel,
        grid=grid,
        in_specs=[
            pl.BlockSpec((BLK, HIDDEN), lambda i: (i, 0)),
            pl.BlockSpec((HIDDEN, N_EXPERTS), lambda i: (0, 0)),
        ],
        out_specs=[
            pl.BlockSpec((BLK, TOPK), lambda i: (i, 0)),
            pl.BlockSpec((BLK, TOPK), lambda i: (i, 0)),
            pl.BlockSpec(memory_space=pltpu.SMEM),
        ],
        out_shape=[
            jax.ShapeDtypeStruct((N_TOKENS, TOPK), jnp.float32),
            jax.ShapeDtypeStruct((N_TOKENS, TOPK), jnp.int32),
            jax.ShapeDtypeStruct((1, 1), jnp.float32),
        ],
        scratch_shapes=[
            pltpu.VMEM((1, N_EXPERTS), jnp.float32),
            pltpu.VMEM((1, N_EXPERTS), jnp.float32),
            pltpu.SMEM((1, 1), jnp.float32),
        ],
    )(hidden_states, wt)
    return weights, indices, aux[0, 0]


# speedup vs baseline: 1.0267x; 1.0267x over previous
import jax
import jax.numpy as jnp
from jax.experimental import pallas as pl
from jax.experimental.pallas import tpu as pltpu

N_TOKENS = 8192
HIDDEN = 2048
N_EXPERTS = 16
TOPK = 2
AUX_COEF = 0.001
Z_COEF = 0.001
BLK = 512


def _router_kernel(x_ref, wt_ref, w_out, i_out, aux_out,
                   cnt_ref, psum_ref, zsum_ref):
    step = pl.program_id(0)
    nsteps = pl.num_programs(0)

    @pl.when(step == 0)
    def _init():
        cnt_ref[...] = jnp.zeros_like(cnt_ref)
        psum_ref[...] = jnp.zeros_like(psum_ref)
        zsum_ref[0, 0] = 0.0

    logits = jnp.dot(x_ref[...], wt_ref[...],
                     preferred_element_type=jnp.float32)
    iota = jax.lax.broadcasted_iota(
        jnp.int32, logits.shape, 1).astype(jnp.float32)

    m1 = jnp.max(logits, axis=1, keepdims=True)
    i1 = jnp.min(jnp.where(logits == m1, iota, float(N_EXPERTS)),
                 axis=1, keepdims=True)
    sel1 = iota == i1
    masked = jnp.where(sel1, -jnp.inf, logits)
    m2 = jnp.max(masked, axis=1, keepdims=True)
    i2 = jnp.min(jnp.where(masked == m2, iota, float(N_EXPERTS)),
                 axis=1, keepdims=True)
    sel2 = iota == i2

    e2 = jnp.exp(m2 - m1)
    w2 = e2 / (1.0 + e2)
    w_out[...] = jnp.concatenate([1.0 - w2, w2], axis=1)
    i_out[...] = jnp.concatenate([i1, i2], axis=1).astype(jnp.int32)

    ex = jnp.exp(logits - m1)
    denom = jnp.sum(ex, axis=1, keepdims=True)
    probs = ex * (1.0 / denom)
    ones_row = jnp.ones((1, probs.shape[0]), dtype=jnp.float32)
    contrib = jnp.where(sel1, 1.0, 0.0) + jnp.where(sel2, 1.0, 0.0)
    cnt_ref[...] += jnp.dot(ones_row, contrib,
                            preferred_element_type=jnp.float32)
    psum_ref[...] += jnp.dot(ones_row, probs,
                             preferred_element_type=jnp.float32)
    log_z = m1 + jnp.log(denom)
    zsum_ref[0, 0] += jnp.sum(log_z * log_z)

    @pl.when(step == nsteps - 1)
    def _fin():
        f = cnt_ref[...] / (N_TOKENS * TOPK)
        p_mean = psum_ref[...] / N_TOKENS
        lb_loss = N_EXPERTS * jnp.sum(f * p_mean)
        z_loss = zsum_ref[0, 0] / N_TOKENS
        aux_out[0, 0] = AUX_COEF * lb_loss + Z_COEF * z_loss


@jax.jit
def kernel(hidden_states, gate_weight):
    wt = gate_weight.T
    grid = (N_TOKENS // BLK,)
    weights, indices, aux = pl.pallas_call(
        _router_kernel,
        grid=grid,
        in_specs=[
            pl.BlockSpec((BLK, HIDDEN), lambda i: (i, 0)),
            pl.BlockSpec((HIDDEN, N_EXPERTS), lambda i: (0, 0)),
        ],
        out_specs=[
            pl.BlockSpec((BLK, TOPK), lambda i: (i, 0)),
            pl.BlockSpec((BLK, TOPK), lambda i: (i, 0)),
            pl.BlockSpec(memory_space=pltpu.SMEM),
        ],
        out_shape=[
            jax.ShapeDtypeStruct((N_TOKENS, TOPK), jnp.float32),
            jax.ShapeDtypeStruct((N_TOKENS, TOPK), jnp.int32),
            jax.ShapeDtypeStruct((1, 1), jnp.float32),
        ],
        scratch_shapes=[
            pltpu.VMEM((1, N_EXPERTS), jnp.float32),
            pltpu.VMEM((1, N_EXPERTS), jnp.float32),
            pltpu.SMEM((1, 1), jnp.float32),
        ],
    )(hidden_states, wt)
    return weights, indices, aux[0, 0]


# speedup vs baseline: 1.1654x; 1.1352x over previous
import jax
import jax.numpy as jnp
from jax.experimental import pallas as pl
from jax.experimental.pallas import tpu as pltpu

N_TOKENS = 8192
HIDDEN = 2048
N_EXPERTS = 16
TOPK = 2
AUX_COEF = 0.001
Z_COEF = 0.001
BLK = 1024


def _router_kernel(x_ref, wt_ref, w_out, i_out, aux_out,
                   cnt_ref, psum_ref, zsum_ref):
    step = pl.program_id(0)
    nsteps = pl.num_programs(0)

    @pl.when(step == 0)
    def _init():
        cnt_ref[...] = jnp.zeros_like(cnt_ref)
        psum_ref[...] = jnp.zeros_like(psum_ref)
        zsum_ref[0, 0] = 0.0

    logits = jnp.dot(x_ref[...], wt_ref[...],
                     preferred_element_type=jnp.float32)
    iota = jax.lax.broadcasted_iota(
        jnp.int32, logits.shape, 1).astype(jnp.float32)

    m1 = jnp.max(logits, axis=1, keepdims=True)
    i1 = jnp.min(jnp.where(logits == m1, iota, float(N_EXPERTS)),
                 axis=1, keepdims=True)
    sel1 = iota == i1
    masked = jnp.where(sel1, -jnp.inf, logits)
    m2 = jnp.max(masked, axis=1, keepdims=True)
    i2 = jnp.min(jnp.where(masked == m2, iota, float(N_EXPERTS)),
                 axis=1, keepdims=True)
    sel2 = iota == i2

    e2 = jnp.exp(m2 - m1)
    w2 = e2 / (1.0 + e2)
    w_out[...] = jnp.concatenate([1.0 - w2, w2], axis=1)
    i_out[...] = jnp.concatenate([i1, i2], axis=1).astype(jnp.int32)

    ex = jnp.exp(logits - m1)
    denom = jnp.sum(ex, axis=1, keepdims=True)
    probs = ex * (1.0 / denom)
    ones_row = jnp.ones((1, probs.shape[0]), dtype=jnp.float32)
    contrib = jnp.where(sel1, 1.0, 0.0) + jnp.where(sel2, 1.0, 0.0)
    cnt_ref[...] += jnp.dot(ones_row, contrib,
                            preferred_element_type=jnp.float32)
    psum_ref[...] += jnp.dot(ones_row, probs,
                             preferred_element_type=jnp.float32)
    log_z = m1 + jnp.log(denom)
    zsum_ref[0, 0] += jnp.sum(log_z * log_z)

    @pl.when(step == nsteps - 1)
    def _fin():
        f = cnt_ref[...] / (N_TOKENS * TOPK)
        p_mean = psum_ref[...] / N_TOKENS
        lb_loss = N_EXPERTS * jnp.sum(f * p_mean)
        z_loss = zsum_ref[0, 0] / N_TOKENS
        aux_out[0, 0] = AUX_COEF * lb_loss + Z_COEF * z_loss


@jax.jit
def kernel(hidden_states, gate_weight):
    wt = gate_weight.T
    grid = (N_TOKENS // BLK,)
    weights, indices, aux = pl.pallas_call(
        _router_kernel,
        grid=grid,
        in_specs=[
            pl.BlockSpec((BLK, HIDDEN), lambda i: (i, 0)),
            pl.BlockSpec((HIDDEN, N_EXPERTS), lambda i: (0, 0)),
        ],
        out_specs=[
            pl.BlockSpec((BLK, TOPK), lambda i: (i, 0)),
            pl.BlockSpec((BLK, TOPK), lambda i: (i, 0)),
            pl.BlockSpec(memory_space=pltpu.SMEM),
        ],
        out_shape=[
            jax.ShapeDtypeStruct((N_TOKENS, TOPK), jnp.float32),
            jax.ShapeDtypeStruct((N_TOKENS, TOPK), jnp.int32),
            jax.ShapeDtypeStruct((1, 1), jnp.float32),
        ],
        scratch_shapes=[
            pltpu.VMEM((1, N_EXPERTS), jnp.float32),
            pltpu.VMEM((1, N_EXPERTS), jnp.float32),
            pltpu.SMEM((1, 1), jnp.float32),
        ],
    )(hidden_states, wt)
    return weights, indices, aux[0, 0]
